# baseline (speedup 1.0000x reference)
import functools

import jax
import jax.numpy as jnp
from jax import lax
from jax.experimental import pallas as pl
from jax.experimental.pallas import tpu as pltpu
from jax.experimental.pallas import tpu_sc as plsc

NUM_EXPERTS = 16
TOP_K = 2
D_MODEL = 2048
D_FF = 8192
TOKENS = 4096

FB1 = 512
TB2 = 512
TBB = 1024
FBB = 2048


def _mm1_body(x_ref, wi_ref, hid_ref):
    wib = wi_ref[...].astype(jnp.bfloat16)
    h = jnp.dot(x_ref[...], wib, preferred_element_type=jnp.float32)
    hid_ref[...] = jnp.maximum(h, 0.0).astype(jnp.bfloat16)


def _score_body(hid_ref, lab_ref, score_ref):
    eids = lax.broadcasted_iota(jnp.int32, (1, NUM_EXPERTS), 1)
    pat_t = (lab_ref[...] == eids).astype(jnp.bfloat16)
    s = jnp.dot(hid_ref[...], pat_t, preferred_element_type=jnp.float32)
    score_ref[...] = s.T


def _routing_sc(score_t):
    info = plsc.get_sparse_core_info()
    nw = info.num_cores * info.num_subcores
    tpw = TOKENS // nw
    groups = tpw // 16
    mesh = plsc.VectorSubcoreMesh(core_axis_name="c", subcore_axis_name="s")

    @functools.partial(
        pl.kernel,
        mesh=mesh,
        out_type=jax.ShapeDtypeStruct((TOKENS,), jnp.int32),
        scratch_types=[
            pltpu.VMEM((NUM_EXPERTS, tpw), jnp.float32),
            pltpu.VMEM((tpw,), jnp.int32),
        ],
    )
    def body(score_hbm, mk_hbm, sc_v, mk_v):
        wid = lax.axis_index("s") * info.num_cores + lax.axis_index("c")
        base = wid * tpw
        pltpu.sync_copy(score_hbm.at[:, pl.ds(base, tpw)], sc_v)
        for g in range(groups):
            t0 = g * 16
            rows = [sc_v[e, pl.ds(t0, 16)] for e in range(NUM_EXPERTS)]
            m1 = rows[0]
            for e in range(1, NUM_EXPERTS):
                m1 = jnp.maximum(m1, rows[e])
            e1 = jnp.full((16,), NUM_EXPERTS, jnp.int32)
            for e in range(NUM_EXPERTS - 1, -1, -1):
                e1 = jnp.where(rows[e] == m1, jnp.int32(e), e1)
            neg = jnp.full((16,), -jnp.inf, jnp.float32)
            rows2 = [jnp.where(e1 == e, neg, rows[e]) for e in range(NUM_EXPERTS)]
            m2 = rows2[0]
            for e in range(1, NUM_EXPERTS):
                m2 = jnp.maximum(m2, rows2[e])
            e2 = jnp.full((16,), NUM_EXPERTS, jnp.int32)
            for e in range(NUM_EXPERTS - 1, -1, -1):
                e2 = jnp.where(rows2[e] == m2, jnp.int32(e), e2)
            one = jnp.full((16,), 1, jnp.int32)
            mk_v[pl.ds(t0, 16)] = (one << e1) | (one << e2)
        pltpu.sync_copy(mk_v, mk_hbm.at[pl.ds(base, tpw)])

    return body(score_t)


def _mm2_body(hid_ref, mk_ref, lab_ref, wo_ref, out_ref):
    j = pl.program_id(1)
    mk = mk_ref[...]
    lab_bit = jnp.int32(1) << lab_ref[...]
    keep = (mk & lab_bit) != 0
    hm = jnp.where(keep, hid_ref[...], jnp.bfloat16(0))
    part = jnp.dot(hm, wo_ref[...], preferred_element_type=jnp.float32)

    @pl.when(j == 0)
    def _():
        out_ref[...] = part

    @pl.when(j != 0)
    def _():
        out_ref[...] += part


def kernel(x, wi_w, wo_w, expert_labels):
    xt = x.reshape(TOKENS, D_MODEL).astype(jnp.bfloat16)
    labels = expert_labels.astype(jnp.int32)

    hidden = pl.pallas_call(
        _mm1_body,
        grid=(D_FF // FB1,),
        in_specs=[
            pl.BlockSpec((TOKENS, D_MODEL), lambda j: (0, 0)),
            pl.BlockSpec((D_MODEL, FB1), lambda j: (0, j)),
        ],
        out_specs=pl.BlockSpec((TOKENS, FB1), lambda j: (0, j)),
        out_shape=jax.ShapeDtypeStruct((TOKENS, D_FF), jnp.bfloat16),
        compiler_params=pltpu.CompilerParams(
            dimension_semantics=("parallel",)),
    )(xt, wi_w)

    score = pl.pallas_call(
        _score_body,
        grid=(TOKENS // TB2,),
        in_specs=[
            pl.BlockSpec((TB2, D_FF), lambda i: (i, 0)),
            pl.BlockSpec((D_FF, 1), lambda i: (0, 0)),
        ],
        out_specs=pl.BlockSpec((NUM_EXPERTS, TB2), lambda i: (0, i)),
        out_shape=jax.ShapeDtypeStruct((NUM_EXPERTS, TOKENS), jnp.float32),
        compiler_params=pltpu.CompilerParams(
            dimension_semantics=("parallel",)),
    )(hidden, labels.reshape(D_FF, 1))

    mk = _routing_sc(score).reshape(TOKENS, 1)

    out = pl.pallas_call(
        _mm2_body,
        grid=(TOKENS // TBB, D_FF // FBB),
        in_specs=[
            pl.BlockSpec((TBB, FBB), lambda t, j: (t, j)),
            pl.BlockSpec((TBB, 1), lambda t, j: (t, 0)),
            pl.BlockSpec((1, FBB), lambda t, j: (0, j)),
            pl.BlockSpec((FBB, D_MODEL), lambda t, j: (j, 0)),
        ],
        out_specs=pl.BlockSpec((TBB, D_MODEL), lambda t, j: (t, 0)),
        out_shape=jax.ShapeDtypeStruct((TOKENS, D_MODEL), jnp.float32),
        compiler_params=pltpu.CompilerParams(
            dimension_semantics=("parallel", "arbitrary")),
    )(hidden, mk, labels.reshape(1, D_FF), wo_w.astype(jnp.bfloat16))

    return out.reshape(x.shape)


# speedup vs baseline: 1.0079x; 1.0079x over previous
import functools

import jax
import jax.numpy as jnp
from jax import lax
from jax.experimental import pallas as pl
from jax.experimental.pallas import tpu as pltpu
from jax.experimental.pallas import tpu_sc as plsc

NUM_EXPERTS = 16
TOP_K = 2
D_MODEL = 2048
D_FF = 8192
TOKENS = 4096

FB1 = 512
TB2 = 1024
TBB = 1024
FBB = 2048


def _mm1_body(x_ref, wi_ref, hid_ref):
    wib = wi_ref[...].astype(jnp.bfloat16)
    for tm in range(TOKENS // 1024):
        sl = pl.ds(tm * 1024, 1024)
        h = jnp.dot(x_ref[sl, :], wib, preferred_element_type=jnp.float32)
        hid_ref[sl, :] = jnp.maximum(h, 0.0).astype(jnp.bfloat16)


def _score_body(hid_ref, lab_ref, score_ref):
    eids = lax.broadcasted_iota(jnp.int32, (1, NUM_EXPERTS), 1)
    pat_t = (lab_ref[...] == eids).astype(jnp.bfloat16)
    s = jnp.dot(hid_ref[...], pat_t, preferred_element_type=jnp.float32)
    score_ref[...] = s.T


def _routing_sc(score_t):
    info = plsc.get_sparse_core_info()
    nw = info.num_cores * info.num_subcores
    tpw = TOKENS // nw
    groups = tpw // 16
    mesh = plsc.VectorSubcoreMesh(core_axis_name="c", subcore_axis_name="s")

    @functools.partial(
        pl.kernel,
        mesh=mesh,
        out_type=jax.ShapeDtypeStruct((TOKENS,), jnp.int32),
        scratch_types=[
            pltpu.VMEM((NUM_EXPERTS, tpw), jnp.float32),
            pltpu.VMEM((tpw,), jnp.int32),
        ],
    )
    def body(score_hbm, mk_hbm, sc_v, mk_v):
        wid = lax.axis_index("s") * info.num_cores + lax.axis_index("c")
        base = wid * tpw
        pltpu.sync_copy(score_hbm.at[:, pl.ds(base, tpw)], sc_v)
        for g in range(groups):
            t0 = g * 16
            rows = [sc_v[e, pl.ds(t0, 16)] for e in range(NUM_EXPERTS)]
            m1 = rows[0]
            for e in range(1, NUM_EXPERTS):
                m1 = jnp.maximum(m1, rows[e])
            e1 = jnp.full((16,), NUM_EXPERTS, jnp.int32)
            for e in range(NUM_EXPERTS - 1, -1, -1):
                e1 = jnp.where(rows[e] == m1, jnp.int32(e), e1)
            neg = jnp.full((16,), -jnp.inf, jnp.float32)
            rows2 = [jnp.where(e1 == e, neg, rows[e]) for e in range(NUM_EXPERTS)]
            m2 = rows2[0]
            for e in range(1, NUM_EXPERTS):
                m2 = jnp.maximum(m2, rows2[e])
            e2 = jnp.full((16,), NUM_EXPERTS, jnp.int32)
            for e in range(NUM_EXPERTS - 1, -1, -1):
                e2 = jnp.where(rows2[e] == m2, jnp.int32(e), e2)
            one = jnp.full((16,), 1, jnp.int32)
            mk_v[pl.ds(t0, 16)] = (one << e1) | (one << e2)
        pltpu.sync_copy(mk_v, mk_hbm.at[pl.ds(base, tpw)])

    return body(score_t)


def _mm2_body(hid_ref, mk_ref, lab_ref, wo_ref, out_ref):
    j = pl.program_id(1)
    mk = mk_ref[...]
    lab_bit = jnp.int32(1) << lab_ref[...]
    keep = (mk & lab_bit) != 0
    hm = jnp.where(keep, hid_ref[...], jnp.bfloat16(0))
    part = jnp.dot(hm, wo_ref[...], preferred_element_type=jnp.float32)

    @pl.when(j == 0)
    def _():
        out_ref[...] = part

    @pl.when(j != 0)
    def _():
        out_ref[...] += part


def kernel(x, wi_w, wo_w, expert_labels):
    xt = x.reshape(TOKENS, D_MODEL).astype(jnp.bfloat16)
    labels = expert_labels.astype(jnp.int32)

    hidden = pl.pallas_call(
        _mm1_body,
        grid=(D_FF // FB1,),
        in_specs=[
            pl.BlockSpec((TOKENS, D_MODEL), lambda j: (0, 0)),
            pl.BlockSpec((D_MODEL, FB1), lambda j: (0, j)),
        ],
        out_specs=pl.BlockSpec((TOKENS, FB1), lambda j: (0, j)),
        out_shape=jax.ShapeDtypeStruct((TOKENS, D_FF), jnp.bfloat16),
        compiler_params=pltpu.CompilerParams(
            dimension_semantics=("parallel",)),
    )(xt, wi_w)

    score = pl.pallas_call(
        _score_body,
        grid=(TOKENS // TB2,),
        in_specs=[
            pl.BlockSpec((TB2, D_FF), lambda i: (i, 0)),
            pl.BlockSpec((D_FF, 1), lambda i: (0, 0)),
        ],
        out_specs=pl.BlockSpec((NUM_EXPERTS, TB2), lambda i: (0, i)),
        out_shape=jax.ShapeDtypeStruct((NUM_EXPERTS, TOKENS), jnp.float32),
        compiler_params=pltpu.CompilerParams(
            dimension_semantics=("parallel",)),
    )(hidden, labels.reshape(D_FF, 1))

    mk = _routing_sc(score).reshape(TOKENS, 1)

    out = pl.pallas_call(
        _mm2_body,
        grid=(TOKENS // TBB, D_FF // FBB),
        in_specs=[
            pl.BlockSpec((TBB, FBB), lambda t, j: (t, j)),
            pl.BlockSpec((TBB, 1), lambda t, j: (t, 0)),
            pl.BlockSpec((1, FBB), lambda t, j: (0, j)),
            pl.BlockSpec((FBB, D_MODEL), lambda t, j: (j, 0)),
        ],
        out_specs=pl.BlockSpec((TBB, D_MODEL), lambda t, j: (t, 0)),
        out_shape=jax.ShapeDtypeStruct((TOKENS, D_MODEL), jnp.float32),
        compiler_params=pltpu.CompilerParams(
            dimension_semantics=("parallel", "arbitrary")),
    )(hidden, mk, labels.reshape(1, D_FF), wo_w.astype(jnp.bfloat16))

    return out.reshape(x.shape)
